# Initial kernel scaffold; baseline (speedup 1.0000x reference)
import jax
import jax.numpy as jnp
from jax.experimental import pallas as pl


def kernel(token_ids, adj, emb, W, b):
    raise NotImplementedError("write your pallas kernel here")


# speedup vs baseline: 1.0073x; 1.0073x over previous
import functools

import jax
import jax.numpy as jnp
from jax import lax
from jax.experimental import pallas as pl
from jax.experimental.pallas import tpu as pltpu
from jax.experimental.pallas import tpu_sc as plsc


_GATHER_CHUNK = 64


@functools.lru_cache(maxsize=None)
def _make_gather(vocab, dim, b_padded):
    info = plsc.get_sparse_core_info()
    nc, ns = info.num_cores, info.num_subcores
    nw = nc * ns
    b_per_w = b_padded // nw
    n_chunks = b_per_w // _GATHER_CHUNK
    mesh = plsc.VectorSubcoreMesh(core_axis_name="c", subcore_axis_name="s")

    @functools.partial(
        pl.kernel,
        mesh=mesh,
        out_type=jax.ShapeDtypeStruct((b_padded, dim), jnp.float32),
        scratch_types=[
            pltpu.VMEM((n_chunks, _GATHER_CHUNK), jnp.int32),
            pltpu.VMEM((b_per_w, dim), jnp.float32),
            pltpu.SemaphoreType.DMA,
        ],
    )
    def gather(table_hbm, idx_hbm, out_hbm, idx_v, rows_v, sem):
        wid = lax.axis_index("s") * nc + lax.axis_index("c")
        base = wid * b_per_w
        for j in range(n_chunks):
            pltpu.sync_copy(
                idx_hbm.at[pl.ds(base + j * _GATHER_CHUNK, _GATHER_CHUNK)],
                idx_v.at[j],
            )
        copies = []
        for j in range(n_chunks):
            copies.append(
                pltpu.async_copy(
                    table_hbm.at[idx_v.at[j]],
                    rows_v.at[pl.ds(j * _GATHER_CHUNK, _GATHER_CHUNK)],
                    sem,
                )
            )
        for c in copies:
            c.wait()
        pltpu.sync_copy(rows_v, out_hbm.at[pl.ds(base, b_per_w)])

    return gather


def _gnn_body(x0_ref, adj_ref, w_ref, b_ref, out_ref, xa, xb, inv):
    layer = pl.program_id(0)
    r = pl.program_id(1)
    blk = adj_ref.shape[0]
    a = adj_ref[...]

    @pl.when(layer == 0)
    def _():
        deg = jnp.sum(a, axis=1, keepdims=True)
        inv[pl.ds(r * blk, blk), :] = 1.0 / (deg + 1e-6)

    iv = inv[pl.ds(r * blk, blk), :]
    wl = w_ref[layer]
    bl = b_ref[layer]

    def step(src_ref):
        x = src_ref[...]
        agg = jnp.dot(a, x, preferred_element_type=jnp.float32) * iv
        lin = jnp.dot(agg, wl, preferred_element_type=jnp.float32) + bl
        return jnp.maximum(lin, 0.0) + src_ref[pl.ds(r * blk, blk), :]

    @pl.when(layer == 0)
    def _():
        xa[pl.ds(r * blk, blk), :] = step(x0_ref)

    @pl.when(layer == 1)
    def _():
        xb[pl.ds(r * blk, blk), :] = step(xa)

    @pl.when(layer == 2)
    def _():
        xa[pl.ds(r * blk, blk), :] = step(xb)

    @pl.when(layer == 3)
    def _():
        h = step(xa)
        m = jnp.max(h, axis=0, keepdims=True)

        @pl.when(r == 0)
        def _():
            out_ref[...] = m

        @pl.when(r > 0)
        def _():
            out_ref[...] = jnp.maximum(out_ref[...], m)


def _gnn(x0, adj, w, b, blk):
    n, dim = x0.shape
    layers = w.shape[0]
    nb = n // blk
    return pl.pallas_call(
        _gnn_body,
        grid=(layers, nb),
        in_specs=[
            pl.BlockSpec((n, dim), lambda l, r: (0, 0)),
            pl.BlockSpec((blk, n), lambda l, r: (r, 0)),
            pl.BlockSpec((layers, dim, dim), lambda l, r: (0, 0, 0)),
            pl.BlockSpec((layers, dim), lambda l, r: (0, 0)),
        ],
        out_specs=pl.BlockSpec((1, dim), lambda l, r: (0, 0)),
        out_shape=jax.ShapeDtypeStruct((1, dim), jnp.float32),
        scratch_shapes=[
            pltpu.VMEM((n, dim), jnp.float32),
            pltpu.VMEM((n, dim), jnp.float32),
            pltpu.VMEM((n, 1), jnp.float32),
        ],
        compiler_params=pltpu.CompilerParams(
            dimension_semantics=("arbitrary", "arbitrary"),
        ),
    )(x0, adj, w, b)


def kernel(token_ids, adj, emb, W, b):
    n = adj.shape[0]
    vocab, dim = emb.shape

    info = plsc.get_sparse_core_info()
    nw = info.num_cores * info.num_subcores
    quantum = nw * _GATHER_CHUNK
    b_padded = ((n + quantum - 1) // quantum) * quantum
    ids = jnp.pad(token_ids.astype(jnp.int32), (0, b_padded - n))
    x0 = _make_gather(vocab, dim, b_padded)(emb, ids)[:n]

    pooled = _gnn(x0, adj, W, b, blk=200)
    return pooled.reshape(dim)
